# baseline (speedup 1.0000x reference)
import functools

import jax
import jax.numpy as jnp
import numpy as np
from jax import lax
from jax.experimental import pallas as pl
from jax.experimental.pallas import tpu as pltpu
from jax.experimental.pallas import tpu_sc as plsc

NC = 2
NS = 16
NW = NC * NS
CH = 128
NBUF = 3


def _sc_scatter_add(e, edge_index, zeros_nh, n_nodes):
    n_edges, h = e.shape
    n_chunks = n_edges // CH
    n_base = n_chunks // NW
    n_extra = n_chunks - n_base * NW
    r_lo = (n_nodes // NS) // 8 * 8
    r_hi = n_nodes - r_lo * (NS - 1)

    mesh = plsc.VectorSubcoreMesh(
        core_axis_name="c", subcore_axis_name="s", num_cores=NC, num_subcores=NS
    )

    @functools.partial(
        pl.kernel,
        out_type=jax.ShapeDtypeStruct((NC, n_nodes, h), jnp.float32),
        mesh=mesh,
        scratch_types=[
            pltpu.VMEM_SHARED((n_nodes, h), jnp.float32),
            pltpu.VMEM((NBUF, CH, h), jnp.float32),
            pltpu.VMEM((NBUF, 2, CH), jnp.int32),
            pltpu.SemaphoreType.DMA,
            pltpu.SemaphoreType.DMA,
            pltpu.SemaphoreType.DMA,
        ],
    )
    def sc_kernel(e_hbm, ei_hbm, zeros_hbm, out_hbm, acc_sh, bbuf, ibuf,
                  lsem, isem, ssem):
        c = lax.axis_index("c")
        s = lax.axis_index("s")
        w = c * NS + s
        n_my = n_base + jnp.where(w < n_extra, 1, 0)

        def start_load(k, buf):
            cid = w + NW * k
            pltpu.async_copy(
                e_hbm.at[pl.ds(cid * CH, CH)], bbuf.at[buf], lsem
            )
            pltpu.async_copy(
                ei_hbm.at[pl.ds(0, 2), pl.ds(cid * CH, CH)], ibuf.at[buf], isem
            )

        def wait_load(buf):
            pltpu.make_async_copy(
                e_hbm.at[pl.ds(0, CH)], bbuf.at[buf], lsem
            ).wait()
            pltpu.make_async_copy(
                ei_hbm.at[pl.ds(0, 2), pl.ds(0, CH)], ibuf.at[buf], isem
            ).wait()

        def wait_scatter_one():
            pltpu.make_async_copy(
                e_hbm.at[pl.ds(0, CH)], bbuf.at[0], ssem
            ).wait()

        start_load(0, 0)
        start_load(1, 1)

        @pl.when(s < NS - 1)
        def _():
            pltpu.sync_copy(
                zeros_hbm.at[pl.ds(s * r_lo, r_lo)],
                acc_sh.at[pl.ds(s * r_lo, r_lo)],
            )

        @pl.when(s == NS - 1)
        def _():
            pltpu.sync_copy(
                zeros_hbm.at[pl.ds((NS - 1) * r_lo, r_hi)],
                acc_sh.at[pl.ds((NS - 1) * r_lo, r_hi)],
            )

        plsc.subcore_barrier()

        def body(k, _):
            buf = lax.rem(k, NBUF)
            wait_load(buf)

            @pl.when(k >= 1)
            def _():
                wait_scatter_one()

            @pl.when(k + 2 < n_my)
            def _():
                start_load(k + 2, lax.rem(k + 2, NBUF))

            pltpu.async_copy(
                bbuf.at[buf], acc_sh.at[ibuf.at[buf, 1]], ssem, add=True
            )
            return 0

        lax.fori_loop(0, n_my, body, 0)
        wait_scatter_one()
        plsc.subcore_barrier()

        @pl.when(s < NS - 1)
        def _():
            pltpu.sync_copy(
                acc_sh.at[pl.ds(s * r_lo, r_lo)],
                out_hbm.at[c, pl.ds(s * r_lo, r_lo)],
            )

        @pl.when(s == NS - 1)
        def _():
            pltpu.sync_copy(
                acc_sh.at[pl.ds((NS - 1) * r_lo, r_hi)],
                out_hbm.at[c, pl.ds((NS - 1) * r_lo, r_hi)],
            )

    return sc_kernel(e, edge_index, zeros_nh)


def _tc_pre(v, W_v, b0, block_rows=2000):
    n, h = v.shape

    def body(v_ref, wv_ref, b0_ref, o_ref):
        o_ref[...] = (
            jnp.dot(v_ref[...], wv_ref[...], preferred_element_type=jnp.float32)
            + b0_ref[...]
        )

    full = lambda i: (0, 0)
    return pl.pallas_call(
        body,
        grid=(n // block_rows,),
        in_specs=[
            pl.BlockSpec((block_rows, h), lambda i: (i, 0)),
            pl.BlockSpec((h, h), full),
            pl.BlockSpec((1, h), full),
        ],
        out_specs=pl.BlockSpec((block_rows, h), lambda i: (i, 0)),
        out_shape=jax.ShapeDtypeStruct((n, h), jnp.float32),
    )(v, W_v, b0.reshape(1, h))


def _tc_mlp(partials, t, W_e, W1, b1, gamma, beta, block_rows=2000):
    n, h = t.shape
    grid = (n // block_rows,)

    def body(p_ref, t_ref, we_ref, w1_ref, b1_ref, g_ref, bt_ref, o_ref):
        agg = p_ref[0] + p_ref[1]
        x = (
            jnp.dot(agg, we_ref[...], preferred_element_type=jnp.float32)
            + t_ref[...]
        )
        x = jnp.maximum(x, 0.0)
        x = jnp.dot(x, w1_ref[...], preferred_element_type=jnp.float32) + b1_ref[...]
        x = jnp.maximum(x, 0.0)
        mu = jnp.mean(x, axis=-1, keepdims=True)
        xc = x - mu
        var = jnp.mean(xc * xc, axis=-1, keepdims=True)
        o_ref[...] = xc * jax.lax.rsqrt(var + 1e-5) * g_ref[...] + bt_ref[...]

    full = lambda i: (0, 0)
    return pl.pallas_call(
        body,
        grid=grid,
        in_specs=[
            pl.BlockSpec((NC, block_rows, h), lambda i: (0, i, 0)),
            pl.BlockSpec((block_rows, h), lambda i: (i, 0)),
            pl.BlockSpec((h, h), full),
            pl.BlockSpec((h, h), full),
            pl.BlockSpec((1, h), full),
            pl.BlockSpec((1, h), full),
            pl.BlockSpec((1, h), full),
        ],
        out_specs=pl.BlockSpec((block_rows, h), lambda i: (i, 0)),
        out_shape=jax.ShapeDtypeStruct((n, h), jnp.float32),
    )(partials, t, W_e, W1, b1.reshape(1, h),
      gamma.reshape(1, h), beta.reshape(1, h))


@jax.jit
def kernel(v, edge_index, e, W_e, W_v, b0, W1, b1, gamma, beta):
    n, h = v.shape
    ei = (edge_index if edge_index.dtype == jnp.int32
          else edge_index.astype(jnp.int32))
    zeros_nh = jnp.zeros((n, h), jnp.float32)
    partials = _sc_scatter_add(e, ei, zeros_nh, n)
    t = _tc_pre(v, W_v, b0)
    return _tc_mlp(partials, t, W_e, W1, b1, gamma, beta)
